# baseline (speedup 1.0000x reference)
import functools

import jax
import jax.numpy as jnp
from jax import lax
from jax.experimental import pallas as pl
from jax.experimental.pallas import tpu as pltpu
from jax.experimental.pallas import tpu_sc as plsc

_F = 26
_V = 100000
_D = 16
_B = 16384
_R = _F * _D
_NC, _NS = 2, 16
_NW = _NC * _NS
_PR = _R // _NW
_OB = 2048
_L = 16
_UN = 8

_mesh = plsc.VectorSubcoreMesh(core_axis_name="c", subcore_axis_name="s")


@functools.partial(
    pl.kernel,
    mesh=_mesh,
    out_type=jax.ShapeDtypeStruct((_R, _B), jnp.float32),
    scratch_types=[
        pltpu.VMEM((_V,), jnp.float32),
        pltpu.VMEM((_B,), jnp.int32),
        pltpu.VMEM((2, _OB), jnp.float32),
        pltpu.SemaphoreType.DMA,
    ],
    compiler_params=pltpu.CompilerParams(needs_layout_passes=False),
)
def _row_gather(idx_hbm, tab_hbm, out_hbm, row_v, idx_v, outb_v, osem):
    wid = lax.axis_index("s") * _NC + lax.axis_index("c")
    r0 = wid * _PR
    nblk = _B // _OB

    def row(k, f_prev):
        r = r0 + k
        f = r // _D
        d = r % _D

        @pl.when(f != f_prev)
        def _():
            pltpu.sync_copy(idx_hbm.at[f], idx_v)

        pltpu.sync_copy(tab_hbm.at[f, d], row_v)

        for b in range(nblk):
            buf = b % 2
            if b >= 2:
                pltpu.make_async_copy(
                    outb_v.at[buf], out_hbm.at[r, pl.ds((b - 2) * _OB, _OB)],
                    osem,
                ).wait()

            @pl.loop(0, _OB // (_L * _UN))
            def step(i):
                base = b * _OB + i * (_L * _UN)
                obase = i * (_L * _UN)
                ivs = [idx_v[pl.ds(base + u * _L, _L)] for u in range(_UN)]
                gs = [plsc.load_gather(row_v, [iv]) for iv in ivs]
                for u in range(_UN):
                    outb_v[buf, pl.ds(obase + u * _L, _L)] = gs[u]

            pltpu.async_copy(
                outb_v.at[buf], out_hbm.at[r, pl.ds(b * _OB, _OB)], osem
            )

        for buf in range(2):
            pltpu.make_async_copy(
                outb_v.at[buf], out_hbm.at[r, pl.ds(0, _OB)], osem
            ).wait()
        return f

    lax.fori_loop(0, _PR, row, -1)


def kernel(onehots, tables):
    idx = onehots.astype(jnp.int32).T
    tab = tables.transpose(0, 2, 1)
    out = _row_gather(idx, tab)
    return out.T
